# baseline (speedup 1.0000x reference)
import functools

import jax
import jax.numpy as jnp
from jax import lax
from jax.experimental import pallas as pl
from jax.experimental.pallas import tpu as pltpu
from jax.experimental.pallas import tpu_sc as plsc

B, F, V, D = 16384, 26, 100000, 16
H1, H2, OUT = 512, 256, 1
VG = V // 8
G = F * VG
NC, NS = 2, 16
NW = NC * NS
BW = B // NW
BWC = 128
NCH = BW // BWC
def _sc_gather(idxT, tabp):
    mesh = plsc.VectorSubcoreMesh(
        core_axis_name="c", subcore_axis_name="s",
        num_cores=NC, num_subcores=NS)

    @functools.partial(
        pl.kernel,
        out_type=jax.ShapeDtypeStruct((B, F * D), jnp.float32),
        mesh=mesh,
        scratch_types=[
            pltpu.VMEM((2, BWC), jnp.int32),
            pltpu.VMEM((2, BWC), jnp.int32),
            pltpu.VMEM((2, BWC, 8 * D), jnp.float32),
            pltpu.VMEM((BWC, F * D), jnp.float32),
            pltpu.SemaphoreType.DMA,
        ],
        compiler_params=pltpu.CompilerParams(use_tc_tiling_on_sc=False,
                                             needs_layout_passes=False),
    )
    def gather_kernel(idx_hbm, tab_hbm, out_hbm, idx_v, gidx_v, rows_v, x_v,
                      sem):
        wid = lax.axis_index("s") * NC + lax.axis_index("c")
        lanes = lax.iota(jnp.int32, 16)

        def load_and_fire(f, sl0, wb):
            pltpu.sync_copy(idx_hbm.at[f, pl.ds(wb, BWC)], idx_v.at[sl0])
            base = f * VG
            for i in range(BWC // 16):
                sl = pl.ds(i * 16, 16)
                gidx_v[sl0, sl] = lax.shift_right_logical(idx_v[sl0, sl], 3) + base
            pltpu.async_copy(tab_hbm.at[gidx_v.at[sl0]], rows_v.at[sl0], sem)

        def drain_and_extract(f, sl0):
            pltpu.make_async_copy(tab_hbm.at[gidx_v.at[sl0]],
                                  rows_v.at[sl0], sem).wait()

            def extract(i, carry3):
                b0 = i * 16
                rowv = b0 + lanes
                m16 = (idx_v[sl0, pl.ds(b0, 16)] & 7) * 16
                for j in range(D):
                    vals = plsc.load_gather(
                        rows_v.at[sl0], [rowv, m16 + j])
                    plsc.store_scatter(
                        x_v, [rowv, jnp.full((16,), f * D, jnp.int32) + j],
                        vals)
                return carry3

            lax.fori_loop(0, BWC // 16, extract, 0)

        def chunk_body(ci, carry):
            wb = wid * BW + ci * BWC
            load_and_fire(0, 0, wb)

            def f_body(f, carry2):
                load_and_fire(f + 1, (f + 1) % 2, wb)
                drain_and_extract(f, f % 2)
                return carry2

            lax.fori_loop(0, F - 1, f_body, 0)
            drain_and_extract(F - 1, (F - 1) % 2)
            pltpu.sync_copy(x_v, out_hbm.at[pl.ds(wb, BWC)])
            return carry

        lax.fori_loop(0, NCH, chunk_body, 0)

    return gather_kernel(idxT, tabp)


TB = 1024


def _mlp_body(x_ref, w1_ref, b1_ref, w2_ref, b2_ref, wo_ref, bo_ref, out_ref):
    h = jnp.dot(x_ref[...], w1_ref[...], preferred_element_type=jnp.float32)
    h = jnp.maximum(h + b1_ref[...], 0.0)
    h = jnp.dot(h, w2_ref[...], preferred_element_type=jnp.float32)
    h = jnp.maximum(h + b2_ref[...], 0.0)
    out_ref[...] = (
        jnp.dot(h, wo_ref[...], preferred_element_type=jnp.float32) + bo_ref[...])


def _mlp(x, W1, b1, W2, b2, Wout, bout):
    return pl.pallas_call(
        _mlp_body,
        grid=(B // TB,),
        in_specs=[
            pl.BlockSpec((TB, F * D), lambda i: (i, 0)),
            pl.BlockSpec((F * D, H1), lambda i: (0, 0)),
            pl.BlockSpec((1, H1), lambda i: (0, 0)),
            pl.BlockSpec((H1, H2), lambda i: (0, 0)),
            pl.BlockSpec((1, H2), lambda i: (0, 0)),
            pl.BlockSpec((H2, OUT), lambda i: (0, 0)),
            pl.BlockSpec((1, OUT), lambda i: (0, 0)),
        ],
        out_specs=pl.BlockSpec((TB, OUT), lambda i: (i, 0)),
        out_shape=jax.ShapeDtypeStruct((B, OUT), jnp.float32),
    )(x, W1, b1, W2, b2, Wout, bout)


def kernel(indices, tables, W1, b1, W2, b2, Wout, bout):
    tabp = tables.reshape(G, 8 * D)
    x = _sc_gather(indices.astype(jnp.int32).T, tabp)
    return _mlp(x, W1, b1.reshape(1, H1), W2, b2.reshape(1, H2),
                Wout, bout.reshape(1, OUT))


# speedup vs baseline: 1.2886x; 1.2886x over previous
import functools

import jax
import jax.numpy as jnp
from jax import lax
from jax.experimental import pallas as pl
from jax.experimental.pallas import tpu as pltpu
from jax.experimental.pallas import tpu_sc as plsc

B, F, V, D = 16384, 26, 100000, 16
H1, H2, OUT = 512, 256, 1
TVV = 25600
NJ = 4
CR = TVV // 8
VG = NJ * CR
G = F * VG
NC, NS = 2, 16
NW = NC * NS
BW = B // NW
BWC = 128
NCH = BW // BWC


def _repack_body(t_ref, out_ref):
    aT = jnp.transpose(t_ref[0], (1, 0))
    a3 = aT.reshape(CR, 8, D)
    for s in range(8):
        out_ref[:, pl.ds(s * D, D)] = a3[:, s, :]


def _tc_repack(tab3):
    return pl.pallas_call(
        _repack_body,
        grid=(F, NJ),
        in_specs=[pl.BlockSpec((1, D, TVV), lambda f, j: (f, 0, j))],
        out_specs=pl.BlockSpec((CR, 8 * D), lambda f, j: (f * NJ + j, 0)),
        out_shape=jax.ShapeDtypeStruct((G, 8 * D), jnp.float32),
    )(tab3)


def _sc_gather(gidxT, m16T, tabp):
    mesh = plsc.VectorSubcoreMesh(
        core_axis_name="c", subcore_axis_name="s",
        num_cores=NC, num_subcores=NS)

    @functools.partial(
        pl.kernel,
        out_type=jax.ShapeDtypeStruct((B, F * D), jnp.float32),
        mesh=mesh,
        scratch_types=[
            pltpu.VMEM((2, BWC), jnp.int32),
            pltpu.VMEM((2, BWC), jnp.int32),
            pltpu.VMEM((2, BWC, 8 * D), jnp.float32),
            pltpu.VMEM((BWC, F * D), jnp.float32),
            pltpu.SemaphoreType.DMA,
        ],
        compiler_params=pltpu.CompilerParams(use_tc_tiling_on_sc=False,
                                             needs_layout_passes=False),
    )
    def gather_kernel(gidx_hbm, m_hbm, tab_hbm, out_hbm, gidx_v, m_v, rows_v,
                      x_v, sem):
        wid = lax.axis_index("s") * NC + lax.axis_index("c")
        lanes = lax.iota(jnp.int32, 16)

        def load_and_fire(f, sl0, wb):
            pltpu.sync_copy(gidx_hbm.at[f, pl.ds(wb, BWC)], gidx_v.at[sl0])
            pltpu.sync_copy(m_hbm.at[f, pl.ds(wb, BWC)], m_v.at[sl0])
            pltpu.async_copy(tab_hbm.at[gidx_v.at[sl0]], rows_v.at[sl0], sem)

        def drain_and_extract(f, sl0):
            pltpu.make_async_copy(tab_hbm.at[gidx_v.at[sl0]],
                                  rows_v.at[sl0], sem).wait()

            def extract(i, carry3):
                b0 = i * 16
                rowv = b0 + lanes
                m16 = m_v[sl0, pl.ds(b0, 16)]
                for j in range(D):
                    vals = plsc.load_gather(
                        rows_v.at[sl0], [rowv, m16 + j])
                    plsc.store_scatter(
                        x_v, [rowv, jnp.full((16,), f * D, jnp.int32) + j],
                        vals)
                return carry3

            lax.fori_loop(0, BWC // 16, extract, 0)

        def chunk_body(ci, carry):
            wb = wid * BW + ci * BWC
            load_and_fire(0, 0, wb)

            def f_body(f, carry2):
                load_and_fire(f + 1, (f + 1) % 2, wb)
                drain_and_extract(f, f % 2)
                return carry2

            lax.fori_loop(0, F - 1, f_body, 0)
            drain_and_extract(F - 1, (F - 1) % 2)
            pltpu.sync_copy(x_v, out_hbm.at[pl.ds(wb, BWC)])
            return carry

        lax.fori_loop(0, NCH, chunk_body, 0)

    return gather_kernel(gidxT, m16T, tabp)


TB = 1024


def _mlp_body(x_ref, w1_ref, b1_ref, w2_ref, b2_ref, wo_ref, bo_ref, out_ref):
    h = jnp.dot(x_ref[...], w1_ref[...], preferred_element_type=jnp.float32)
    h = jnp.maximum(h + b1_ref[...], 0.0)
    h = jnp.dot(h, w2_ref[...], preferred_element_type=jnp.float32)
    h = jnp.maximum(h + b2_ref[...], 0.0)
    out_ref[...] = (
        jnp.dot(h, wo_ref[...], preferred_element_type=jnp.float32) + bo_ref[...])


def _mlp(x, W1, b1, W2, b2, Wout, bout):
    return pl.pallas_call(
        _mlp_body,
        grid=(B // TB,),
        in_specs=[
            pl.BlockSpec((TB, F * D), lambda i: (i, 0)),
            pl.BlockSpec((F * D, H1), lambda i: (0, 0)),
            pl.BlockSpec((1, H1), lambda i: (0, 0)),
            pl.BlockSpec((H1, H2), lambda i: (0, 0)),
            pl.BlockSpec((1, H2), lambda i: (0, 0)),
            pl.BlockSpec((H2, OUT), lambda i: (0, 0)),
            pl.BlockSpec((1, OUT), lambda i: (0, 0)),
        ],
        out_specs=pl.BlockSpec((TB, OUT), lambda i: (i, 0)),
        out_shape=jax.ShapeDtypeStruct((B, OUT), jnp.float32),
    )(x, W1, b1, W2, b2, Wout, bout)


def kernel(indices, tables, W1, b1, W2, b2, Wout, bout):
    tab3 = jnp.swapaxes(tables, 1, 2)
    tabp = _tc_repack(tab3)
    idxT = indices.astype(jnp.int32).T
    fbase = (jnp.arange(F, dtype=jnp.int32) * VG)[:, None]
    gidxT = fbase + (idxT >> 3)
    m16T = (idxT & 7) * D
    x = _sc_gather(gidxT, m16T, tabp)
    return _mlp(x, W1, b1.reshape(1, H1), W2, b2.reshape(1, H2),
                Wout, bout.reshape(1, OUT))


# speedup vs baseline: 2.0769x; 1.6117x over previous
import functools

import jax
import jax.numpy as jnp
from jax import lax
from jax.experimental import pallas as pl
from jax.experimental.pallas import tpu as pltpu
from jax.experimental.pallas import tpu_sc as plsc

B, F, V, D = 16384, 26, 100000, 16
H1, H2, OUT = 512, 256, 1
BF = B * F
NC, NS = 2, 16
NW = NC * NS
BW = B // NW
BWC = 256
NCH = BW // BWC


FH = F // 2


def _sc_gather(idxT, tab_flat):
    mesh = plsc.VectorSubcoreMesh(
        core_axis_name="c", subcore_axis_name="s",
        num_cores=NC, num_subcores=NS)

    @functools.partial(
        pl.kernel,
        out_type=jax.ShapeDtypeStruct((B, FH * D), jnp.float32),
        mesh=mesh,
        scratch_types=[
            pltpu.VMEM((2, BWC), jnp.int32),
            pltpu.VMEM((2, D, BWC), jnp.int32),
            pltpu.VMEM((2, D, BWC), jnp.float32),
            pltpu.VMEM((BWC, FH * D), jnp.float32),
            pltpu.SemaphoreType.DMA,
        ],
        compiler_params=pltpu.CompilerParams(use_tc_tiling_on_sc=False,
                                             needs_layout_passes=False),
    )
    def gather_kernel(idx_hbm, tab_hbm, out_hbm, idx_v, fidx_v, rows_v, x_v,
                      sem):
        wid = lax.axis_index("s") * NC + lax.axis_index("c")
        lanes = lax.iota(jnp.int32, 16)

        def load_and_fire(f, sl0, wb):
            pltpu.sync_copy(idx_hbm.at[f, pl.ds(wb, BWC)], idx_v.at[sl0])

            def fire(d, carry3):
                base = f * (D * V) + d * V
                for i in range(BWC // 16):
                    sl = pl.ds(i * 16, 16)
                    fidx_v[sl0, d, sl] = idx_v[sl0, sl] + base
                pltpu.async_copy(tab_hbm.at[fidx_v.at[sl0, d]],
                                 rows_v.at[sl0, d], sem)
                return carry3

            lax.fori_loop(0, D, fire, 0)

        def drain_and_scatter(f, sl0):
            def drain(d, carry3):
                pltpu.make_async_copy(tab_hbm.at[fidx_v.at[sl0, d]],
                                      rows_v.at[sl0, d], sem).wait()
                return carry3

            lax.fori_loop(0, D, drain, 0)

            def scatter(d, carry3):
                col = jnp.full((16,), f * D, jnp.int32) + d
                for i in range(BWC // 16):
                    vals = rows_v[sl0, d, pl.ds(i * 16, 16)]
                    plsc.store_scatter(x_v, [i * 16 + lanes, col], vals)
                return carry3

            lax.fori_loop(0, D, scatter, 0)

        def chunk_body(ci, carry):
            wb = wid * BW + ci * BWC
            load_and_fire(0, 0, wb)

            def f_body(f, carry2):
                load_and_fire(f + 1, (f + 1) % 2, wb)
                drain_and_scatter(f, f % 2)
                return carry2

            lax.fori_loop(0, FH - 1, f_body, 0)
            drain_and_scatter(FH - 1, (FH - 1) % 2)
            pltpu.sync_copy(x_v, out_hbm.at[pl.ds(wb, BWC)])
            return carry

        lax.fori_loop(0, NCH, chunk_body, 0)

    return gather_kernel(idxT, tab_flat)


TB = 1024


def _mlp_body(x_ref, w1_ref, b1_ref, w2_ref, b2_ref, wo_ref, bo_ref, out_ref):
    h = jnp.dot(x_ref[...], w1_ref[...], preferred_element_type=jnp.float32)
    h = jnp.maximum(h + b1_ref[...], 0.0)
    h = jnp.dot(h, w2_ref[...], preferred_element_type=jnp.float32)
    h = jnp.maximum(h + b2_ref[...], 0.0)
    out_ref[...] = (
        jnp.dot(h, wo_ref[...], preferred_element_type=jnp.float32) + bo_ref[...])


def _mlp(x, W1, b1, W2, b2, Wout, bout):
    return pl.pallas_call(
        _mlp_body,
        grid=(B // TB,),
        in_specs=[
            pl.BlockSpec((TB, F * D), lambda i: (i, 0)),
            pl.BlockSpec((F * D, H1), lambda i: (0, 0)),
            pl.BlockSpec((1, H1), lambda i: (0, 0)),
            pl.BlockSpec((H1, H2), lambda i: (0, 0)),
            pl.BlockSpec((1, H2), lambda i: (0, 0)),
            pl.BlockSpec((H2, OUT), lambda i: (0, 0)),
            pl.BlockSpec((1, OUT), lambda i: (0, 0)),
        ],
        out_specs=pl.BlockSpec((TB, OUT), lambda i: (i, 0)),
        out_shape=jax.ShapeDtypeStruct((B, OUT), jnp.float32),
    )(x, W1, b1, W2, b2, Wout, bout)


def kernel(indices, tables, W1, b1, W2, b2, Wout, bout):
    idxT = indices.astype(jnp.int32).T
    tabs = jnp.swapaxes(tables, 1, 2)
    xs = [_sc_gather(idxT[h * FH:(h + 1) * FH],
                     tabs[h * FH:(h + 1) * FH].reshape(FH * D * V))
          for h in range(2)]
    x = jnp.concatenate(xs, axis=1)
    return _mlp(x, W1, b1.reshape(1, H1), W2, b2.reshape(1, H2),
                Wout, bout.reshape(1, OUT))


# speedup vs baseline: 2.1537x; 1.0370x over previous
import functools

import jax
import jax.numpy as jnp
from jax import lax
from jax.experimental import pallas as pl
from jax.experimental.pallas import tpu as pltpu
from jax.experimental.pallas import tpu_sc as plsc

B, F, V, D = 16384, 26, 100000, 16
H1, H2, OUT = 512, 256, 1
BF = B * F
NC, NS = 2, 16
NW = NC * NS
BW = B // NW
BWC = 256
NCH = BW // BWC


FH = F // 2


def _sc_gather(idxT, tab_flat):
    mesh = plsc.VectorSubcoreMesh(
        core_axis_name="c", subcore_axis_name="s",
        num_cores=NC, num_subcores=NS)

    @functools.partial(
        pl.kernel,
        out_type=jax.ShapeDtypeStruct((B, FH * D), jnp.float32),
        mesh=mesh,
        scratch_types=[
            pltpu.VMEM((2, BWC), jnp.int32),
            pltpu.VMEM((2, D, BWC), jnp.int32),
            pltpu.VMEM((2, D, BWC), jnp.float32),
            pltpu.VMEM((BWC, FH * D), jnp.float32),
            pltpu.SemaphoreType.DMA,
        ],
        compiler_params=pltpu.CompilerParams(use_tc_tiling_on_sc=False,
                                             needs_layout_passes=False),
    )
    def gather_kernel(idx_hbm, tab_hbm, out_hbm, idx_v, fidx_v, rows_v, x_v,
                      sem):
        wid = lax.axis_index("s") * NC + lax.axis_index("c")
        lanes = lax.iota(jnp.int32, 16)

        def load_and_fire(f, sl0, wb):
            pltpu.sync_copy(idx_hbm.at[f, pl.ds(wb, BWC)], idx_v.at[sl0])

            def fire(d, carry3):
                base = f * (D * V) + d * V
                for i in range(BWC // 16):
                    sl = pl.ds(i * 16, 16)
                    fidx_v[sl0, d, sl] = idx_v[sl0, sl] + base
                pltpu.async_copy(tab_hbm.at[fidx_v.at[sl0, d]],
                                 rows_v.at[sl0, d], sem)
                return carry3

            lax.fori_loop(0, D, fire, 0)

        def drain_and_scatter(f, sl0):
            def drain(d, carry3):
                pltpu.make_async_copy(tab_hbm.at[fidx_v.at[sl0, d]],
                                      rows_v.at[sl0, d], sem).wait()
                return carry3

            lax.fori_loop(0, D, drain, 0)

            def scatter(d, carry3):
                col = jnp.full((16,), f * D, jnp.int32) + d
                for i in range(BWC // 16):
                    vals = rows_v[sl0, d, pl.ds(i * 16, 16)]
                    plsc.store_scatter(x_v, [i * 16 + lanes, col], vals)
                return carry3

            lax.fori_loop(0, D, scatter, 0)

        def chunk_body(ci, carry):
            wb = wid * BW + ci * BWC
            load_and_fire(0, 0, wb)

            def f_body(f, carry2):
                load_and_fire(f + 1, (f + 1) % 2, wb)
                drain_and_scatter(f, f % 2)
                return carry2

            lax.fori_loop(0, FH - 1, f_body, 0)
            drain_and_scatter(FH - 1, (FH - 1) % 2)
            pltpu.sync_copy(x_v, out_hbm.at[pl.ds(wb, BWC)])
            return carry

        lax.fori_loop(0, NCH, chunk_body, 0)

    return gather_kernel(idxT, tab_flat)


TB = 1024


def _mlp_body(xa_ref, xb_ref, w1a_ref, w1b_ref, b1_ref, w2_ref, b2_ref,
              wo_ref, bo_ref, out_ref):
    h = (jnp.dot(xa_ref[...], w1a_ref[...], preferred_element_type=jnp.float32)
         + jnp.dot(xb_ref[...], w1b_ref[...],
                   preferred_element_type=jnp.float32))
    h = jnp.maximum(h + b1_ref[...], 0.0)
    h = jnp.dot(h, w2_ref[...], preferred_element_type=jnp.float32)
    h = jnp.maximum(h + b2_ref[...], 0.0)
    out_ref[...] = (
        jnp.dot(h, wo_ref[...], preferred_element_type=jnp.float32) + bo_ref[...])


def _mlp(xa, xb, W1, b1, W2, b2, Wout, bout):
    return pl.pallas_call(
        _mlp_body,
        grid=(B // TB,),
        in_specs=[
            pl.BlockSpec((TB, FH * D), lambda i: (i, 0)),
            pl.BlockSpec((TB, FH * D), lambda i: (i, 0)),
            pl.BlockSpec((FH * D, H1), lambda i: (0, 0)),
            pl.BlockSpec((FH * D, H1), lambda i: (0, 0)),
            pl.BlockSpec((1, H1), lambda i: (0, 0)),
            pl.BlockSpec((H1, H2), lambda i: (0, 0)),
            pl.BlockSpec((1, H2), lambda i: (0, 0)),
            pl.BlockSpec((H2, OUT), lambda i: (0, 0)),
            pl.BlockSpec((1, OUT), lambda i: (0, 0)),
        ],
        out_specs=pl.BlockSpec((TB, OUT), lambda i: (i, 0)),
        out_shape=jax.ShapeDtypeStruct((B, OUT), jnp.float32),
    )(xa, xb, W1[:FH * D], W1[FH * D:], b1, W2, b2, Wout, bout)


def kernel(indices, tables, W1, b1, W2, b2, Wout, bout):
    idxT = indices.astype(jnp.int32).T
    xs = [_sc_gather(idxT[h * FH:(h + 1) * FH],
                     jnp.swapaxes(tables[h * FH:(h + 1) * FH], 1, 2)
                     .reshape(FH * D * V))
          for h in range(2)]
    return _mlp(xs[0], xs[1], W1, b1.reshape(1, H1), W2, b2.reshape(1, H2),
                Wout, bout.reshape(1, OUT))
